# baseline (speedup 1.0000x reference)
import functools

import jax
import jax.numpy as jnp
from jax import lax
from jax.experimental import pallas as pl
from jax.experimental.pallas import tpu as pltpu
from jax.experimental.pallas import tpu_sc as plsc

N = 10000
VOCAB = 100000
D = 128
NH = 2
FH = 64
ALPHA = 0.2
LOG2E = 1.4426950408889634

BPAD = 10240
NWORKERS = 32
BPW = BPAD // NWORKERS
GCH = 80

BI = 200
BP = 1000
WX = 2 * D


def _sc_gather(ids_pad, table):
    mesh = plsc.VectorSubcoreMesh(core_axis_name="c", subcore_axis_name="s")

    @functools.partial(
        pl.kernel,
        mesh=mesh,
        out_type=jax.ShapeDtypeStruct((BPAD, D), jnp.float32),
        scratch_types=[
            pltpu.VMEM((BPW,), jnp.int32),
            pltpu.VMEM((BPW, D), jnp.float32),
            pltpu.SemaphoreType.DMA,
        ],
    )
    def gat_emb_gather(idx_hbm, table_hbm, out_hbm, idx_v, rows_v, sem):
        wid = lax.axis_index("s") * 2 + lax.axis_index("c")
        base = wid * BPW
        pltpu.sync_copy(idx_hbm.at[pl.ds(base, BPW)], idx_v)
        copies = []
        for off, ln in ((0, 128), (128, 128), (256, 64)):
            copies.append(
                pltpu.async_copy(
                    table_hbm.at[idx_v.at[pl.ds(off, ln)]],
                    rows_v.at[pl.ds(off, ln)],
                    sem,
                )
            )
        for cp in copies:
            cp.wait()
        pltpu.sync_copy(rows_v, out_hbm.at[pl.ds(base, BPW)])

    return gat_emb_gather(ids_pad, table)


def _proj_body(h_ref, wpad_ref, ones_ref, wa_ref, whx_ref, f_ref, wsum_ref):
    i = pl.program_id(0)
    whx = jnp.dot(h_ref[...], wpad_ref[...],
                  preferred_element_type=jnp.float32) + ones_ref[...]
    whx_ref[...] = whx
    f_ref[...] = jnp.dot(h_ref[...], wa_ref[...],
                         preferred_element_type=jnp.float32)
    csum = jnp.sum(whx, axis=0, keepdims=True)

    @pl.when(i == 0)
    def _():
        wsum_ref[...] = csum

    @pl.when(i > 0)
    def _():
        wsum_ref[...] = wsum_ref[...] + csum


def _proj(h, wpad, onesrow, wa):
    return pl.pallas_call(
        _proj_body,
        grid=(N // BP,),
        in_specs=[
            pl.BlockSpec((BP, D), lambda i: (i, 0)),
            pl.BlockSpec((D, WX), lambda i: (0, 0)),
            pl.BlockSpec((1, WX), lambda i: (0, 0)),
            pl.BlockSpec((D, 8), lambda i: (0, 0)),
        ],
        out_specs=[
            pl.BlockSpec((BP, WX), lambda i: (i, 0)),
            pl.BlockSpec((BP, 8), lambda i: (i, 0)),
            pl.BlockSpec((1, WX), lambda i: (0, 0)),
        ],
        out_shape=[
            jax.ShapeDtypeStruct((N, WX), jnp.float32),
            jax.ShapeDtypeStruct((N, 8), jnp.float32),
            jax.ShapeDtypeStruct((1, WX), jnp.float32),
        ],
        compiler_params=pltpu.CompilerParams(
            dimension_semantics=("arbitrary",)),
    )(h, wpad, onesrow, wa)


def _att_body(adj_ref, fs_ref, fdt_ref, whx_ref, wsum_ref, *rest, last):
    if last:
        out_ref, mask_ref = rest
    else:
        hin_ref, out_ref, mask_ref = rest

    def head_p(h, maskf):
        fs = fs_ref[:, h:h + 1]
        fd = fdt_ref[2 + h:3 + h, :]
        mfd = jnp.max(fd, axis=1, keepdims=True)
        yb = fs + mfd
        mb = jnp.maximum(yb, ALPHA * yb)
        ed1 = jnp.exp2(fd - mfd)
        ed2 = jnp.exp2(ALPHA * (fd - mfd))
        es1 = jnp.exp2(yb - mb)
        es2 = jnp.exp2(ALPHA * yb - mb)
        p = jnp.maximum(es1 * ed1, es2 * ed2)
        if maskf is not None:
            p = p * maskf
        return p

    def epilogue(outs):
        if last:
            out_ref[...] = 0.5 * (outs[0] + outs[1])
        else:
            oc = jnp.concatenate(outs, axis=1)
            oc = jnp.where(oc > 0.0, oc, jnp.exp(oc) - 1.0)
            out_ref[...] = hin_ref[...] + oc

    mask_ref[...] = jnp.ceil(adj_ref[...])
    maskf = mask_ref[...]
    outs = []
    for h in range(NH):
        p = head_p(h, maskf)
        mm = jnp.dot(p, whx_ref[:, D * h:D * h + D],
                     preferred_element_type=jnp.float32)
        l = mm[:, FH:FH + 1]
        o = jnp.where(l > 0.0,
                      mm[:, :FH] * (1.0 / l),
                      wsum_ref[:, D * h:D * h + FH] * (1.0 / N))
        outs.append(o)
    epilogue(outs)


def _att_layer(adj, f, fdt, whx, wsum, hin, *, last):
    in_specs = [
        pl.BlockSpec((BI, N), lambda i: (i, 0)),
        pl.BlockSpec((BI, 8), lambda i: (i, 0)),
        pl.BlockSpec((8, N), lambda i: (0, 0)),
        pl.BlockSpec((N, WX), lambda i: (0, 0)),
        pl.BlockSpec((1, WX), lambda i: (0, 0)),
    ]
    inputs = [adj, f, fdt, whx, wsum]
    if not last:
        in_specs.append(pl.BlockSpec((BI, D), lambda i: (i, 0)))
        inputs.append(hin)
    od = FH if last else D
    return pl.pallas_call(
        functools.partial(_att_body, last=last),
        grid=(N // BI,),
        in_specs=in_specs,
        out_specs=pl.BlockSpec((BI, od), lambda i: (i, 0)),
        out_shape=jax.ShapeDtypeStruct((N, od), jnp.float32),
        scratch_shapes=[pltpu.VMEM((BI, N), jnp.float32)],
        compiler_params=pltpu.CompilerParams(
            dimension_semantics=("parallel",)),
    )(*inputs)


def _pack_weights(W0, W1, a):
    wpad = jnp.zeros((D, WX), jnp.float32)
    wpad = wpad.at[:, :FH].set(W0)
    wpad = wpad.at[:, D:D + FH].set(W1)
    onesrow = jnp.zeros((1, WX), jnp.float32)
    onesrow = onesrow.at[0, FH].set(1.0)
    onesrow = onesrow.at[0, D + FH].set(1.0)
    wa = jnp.stack(
        [W0 @ a[0, 0], W1 @ a[1, 0], W0 @ a[0, 1], W1 @ a[1, 1]] +
        [jnp.zeros((D,), jnp.float32)] * 4, axis=1) * LOG2E
    return wpad, onesrow, wa


def kernel(tcword_id, adj, emb_table, W_mid, a_mid, W_last, a_last):
    ids = jnp.pad(tcword_id.astype(jnp.int32), (0, BPAD - N))
    h0 = _sc_gather(ids, emb_table)

    wpad1, ones1, wa1 = _pack_weights(W_mid[0, 0], W_mid[0, 1], a_mid[0])
    whx1, f1, wsum1 = _proj(h0, wpad1, ones1, wa1)
    h1 = _att_layer(adj, f1, f1.T, whx1, wsum1, h0, last=False)

    wpad2, ones2, wa2 = _pack_weights(W_last[0], W_last[1], a_last)
    whx2, f2, wsum2 = _proj(h1, wpad2, ones2, wa2)
    return _att_layer(adj, f2, f2.T, whx2, wsum2, None, last=True)


# speedup vs baseline: 1.0030x; 1.0030x over previous
import functools

import jax
import jax.numpy as jnp
from jax import lax
from jax.experimental import pallas as pl
from jax.experimental.pallas import tpu as pltpu
from jax.experimental.pallas import tpu_sc as plsc

N = 10000
VOCAB = 100000
D = 128
NH = 2
FH = 64
ALPHA = 0.2
LOG2E = 1.4426950408889634

BPAD = 10240
NWORKERS = 32
BPW = BPAD // NWORKERS

BI = 200
BP = 1000
WX = 2 * D


def _sc_gather(ids_pad, table):
    mesh = plsc.VectorSubcoreMesh(core_axis_name="c", subcore_axis_name="s")

    @functools.partial(
        pl.kernel,
        mesh=mesh,
        out_type=jax.ShapeDtypeStruct((BPAD, D), jnp.float32),
        scratch_types=[
            pltpu.VMEM((BPW,), jnp.int32),
            pltpu.VMEM((BPW, D), jnp.float32),
            pltpu.SemaphoreType.DMA,
        ],
    )
    def gat_emb_gather(idx_hbm, table_hbm, out_hbm, idx_v, rows_v, sem):
        wid = lax.axis_index("s") * 2 + lax.axis_index("c")
        base = wid * BPW
        pltpu.sync_copy(idx_hbm.at[pl.ds(base, BPW)], idx_v)
        copies = []
        for off, ln in ((0, 128), (128, 128), (256, 64)):
            copies.append(
                pltpu.async_copy(
                    table_hbm.at[idx_v.at[pl.ds(off, ln)]],
                    rows_v.at[pl.ds(off, ln)],
                    sem,
                )
            )
        for cp in copies:
            cp.wait()
        pltpu.sync_copy(rows_v, out_hbm.at[pl.ds(base, BPW)])

    return gat_emb_gather(ids_pad, table)


def _proj_body(h_ref, wpad_ref, ones_ref, wa_ref, whx_ref, f_ref, wsum_ref):
    i = pl.program_id(0)
    whx = jnp.dot(h_ref[...], wpad_ref[...],
                  preferred_element_type=jnp.float32) + ones_ref[...]
    whx_ref[...] = whx
    f_ref[...] = jnp.dot(h_ref[...], wa_ref[...],
                         preferred_element_type=jnp.float32)
    csum = jnp.sum(whx, axis=0, keepdims=True)

    @pl.when(i == 0)
    def _():
        wsum_ref[...] = csum

    @pl.when(i > 0)
    def _():
        wsum_ref[...] = wsum_ref[...] + csum


def _proj(h, wpad, onesrow, wa):
    return pl.pallas_call(
        _proj_body,
        grid=(N // BP,),
        in_specs=[
            pl.BlockSpec((BP, D), lambda i: (i, 0)),
            pl.BlockSpec((D, WX), lambda i: (0, 0)),
            pl.BlockSpec((1, WX), lambda i: (0, 0)),
            pl.BlockSpec((D, 8), lambda i: (0, 0)),
        ],
        out_specs=[
            pl.BlockSpec((BP, WX), lambda i: (i, 0)),
            pl.BlockSpec((BP, 8), lambda i: (i, 0)),
            pl.BlockSpec((1, WX), lambda i: (0, 0)),
        ],
        out_shape=[
            jax.ShapeDtypeStruct((N, WX), jnp.float32),
            jax.ShapeDtypeStruct((N, 8), jnp.float32),
            jax.ShapeDtypeStruct((1, WX), jnp.float32),
        ],
        compiler_params=pltpu.CompilerParams(
            dimension_semantics=("arbitrary",)),
    )(h, wpad, onesrow, wa)


def _att_body(adj_ref, fs_ref, fdt_ref, whx_ref, wsum_ref, *rest, last):
    if last:
        out_ref, mask_ref = rest
    else:
        hin_ref, out_ref, mask_ref = rest

    def head_p(h, maskf):
        fs = fs_ref[:, h:h + 1]
        fd = fdt_ref[2 + h:3 + h, :]
        mfd = jnp.max(fd, axis=1, keepdims=True)
        yb = fs + mfd
        mb = jnp.maximum(yb, ALPHA * yb)
        ed1 = jnp.exp2(fd - mfd)
        ed2 = jnp.exp2(ALPHA * (fd - mfd))
        es1 = jnp.exp2(yb - mb)
        es2 = jnp.exp2(ALPHA * yb - mb)
        p = jnp.maximum(es1 * ed1, es2 * ed2)
        if maskf is not None:
            p = p * maskf
        return p

    def epilogue(outs):
        if last:
            out_ref[...] = 0.5 * (outs[0] + outs[1])
        else:
            oc = jnp.concatenate(outs, axis=1)
            oc = jnp.where(oc > 0.0, oc, jnp.exp(oc) - 1.0)
            out_ref[...] = hin_ref[...] + oc

    mask_ref[...] = jnp.ceil(adj_ref[...])
    maskf = mask_ref[...]
    outs = []
    for h in range(NH):
        p = head_p(h, maskf)
        mm = jnp.dot(p, whx_ref[:, D * h:D * h + D],
                     preferred_element_type=jnp.float32)
        l = mm[:, FH:FH + 1]
        o = jnp.where(l > 0.0,
                      mm[:, :FH] * (1.0 / l),
                      wsum_ref[:, D * h:D * h + FH] * (1.0 / N))
        outs.append(o)
    epilogue(outs)


def _att_layer(adj, f, fdt, whx, wsum, hin, *, last):
    in_specs = [
        pl.BlockSpec((BI, N), lambda i: (i, 0)),
        pl.BlockSpec((BI, 8), lambda i: (i, 0)),
        pl.BlockSpec((8, N), lambda i: (0, 0)),
        pl.BlockSpec((N, WX), lambda i: (0, 0)),
        pl.BlockSpec((1, WX), lambda i: (0, 0)),
    ]
    inputs = [adj, f, fdt, whx, wsum]
    if not last:
        in_specs.append(pl.BlockSpec((BI, D), lambda i: (i, 0)))
        inputs.append(hin)
    od = FH if last else D
    return pl.pallas_call(
        functools.partial(_att_body, last=last),
        grid=(N // BI,),
        in_specs=in_specs,
        out_specs=pl.BlockSpec((BI, od), lambda i: (i, 0)),
        out_shape=jax.ShapeDtypeStruct((N, od), jnp.float32),
        scratch_shapes=[pltpu.VMEM((BI, N), jnp.float32)],
        compiler_params=pltpu.CompilerParams(
            dimension_semantics=("parallel",)),
    )(*inputs)


def _pack_weights(W0, W1, a):
    wpad = jnp.zeros((D, WX), jnp.float32)
    wpad = wpad.at[:, :FH].set(W0)
    wpad = wpad.at[:, D:D + FH].set(W1)
    onesrow = jnp.zeros((1, WX), jnp.float32)
    onesrow = onesrow.at[0, FH].set(1.0)
    onesrow = onesrow.at[0, D + FH].set(1.0)
    wa = jnp.stack(
        [W0 @ a[0, 0], W1 @ a[1, 0], W0 @ a[0, 1], W1 @ a[1, 1]] +
        [jnp.zeros((D,), jnp.float32)] * 4, axis=1) * LOG2E
    return wpad, onesrow, wa


def kernel(tcword_id, adj, emb_table, W_mid, a_mid, W_last, a_last):
    ids = jnp.pad(tcword_id.astype(jnp.int32), (0, BPAD - N))
    h0 = _sc_gather(ids, emb_table)

    wpad1, ones1, wa1 = _pack_weights(W_mid[0, 0], W_mid[0, 1], a_mid[0])
    whx1, f1, wsum1 = _proj(h0, wpad1, ones1, wa1)
    h1 = _att_layer(adj, f1, f1.T, whx1, wsum1, h0, last=False)

    wpad2, ones2, wa2 = _pack_weights(W_last[0], W_last[1], a_last)
    whx2, f2, wsum2 = _proj(h1, wpad2, ones2, wa2)
    return _att_layer(adj, f2, f2.T, whx2, wsum2, None, last=True)
